# baseline (speedup 1.0000x reference)
import functools

import jax
import jax.numpy as jnp
from jax import lax
from jax.experimental import pallas as pl
from jax.experimental.pallas import tpu as pltpu
from jax.experimental.pallas import tpu_sc as plsc

NC = 2
NS = 16
NW = NC * NS
CHUNK = 128


def _sc_agg(x2d, edge_index):
    n, d = x2d.shape
    e = edge_index.shape[1]
    assert e % CHUNK == 0
    ncht = e // CHUNK
    K = ncht // NW
    L = ncht - NW * K
    rows_per = (n // NS) // 8 * 8
    extra = n - NS * rows_per
    nz = rows_per // CHUNK
    remz = rows_per - nz * CHUNK

    mesh = plsc.VectorSubcoreMesh(core_axis_name="c", subcore_axis_name="s")

    @functools.partial(
        pl.kernel,
        out_type=jax.ShapeDtypeStruct((NC, n, d), jnp.float32),
        mesh=mesh,
        scratch_types=[
            pltpu.VMEM((2, CHUNK), jnp.int32),
            pltpu.VMEM((2, CHUNK), jnp.int32),
            pltpu.VMEM((2, CHUNK), jnp.int32),
            pltpu.VMEM((2, CHUNK), jnp.int32),
            pltpu.VMEM((CHUNK, d), jnp.float32),
            pltpu.VMEM((CHUNK, d), jnp.float32),
            pltpu.VMEM_SHARED((n, d), jnp.float32),
            pltpu.SemaphoreType.DMA,
            pltpu.SemaphoreType.DMA,
            pltpu.SemaphoreType.DMA,
            pltpu.SemaphoreType.DMA,
            pltpu.SemaphoreType.DMA,
            pltpu.SemaphoreType.DMA,
        ],
    )
    def k(x_hbm, ei_hbm, out_hbm,
          ib0, ib1, ib2, ib3, r0, r1, acc,
          sb0, sb1, sb2, sb3, sg0, sg1):
        ib = [ib0, ib1, ib2, ib3]
        sb = [sb0, sb1, sb2, sb3]
        r = [r0, r1]
        sg = [sg0, sg1]
        cid = lax.axis_index("c")
        sid = lax.axis_index("s")
        wid = cid * NS + sid

        def col(kk):
            return (wid + NW * kk) * CHUNK

        def idx_issue(kk, j):
            pltpu.async_copy(
                ei_hbm.at[pl.ds(0, 2), pl.ds(col(kk), CHUNK)], ib[j], sb[j])

        def idx_wait(j):
            pltpu.make_async_copy(
                ei_hbm.at[pl.ds(0, 2), pl.ds(0, CHUNK)], ib[j], sb[j]).wait()

        def gather_issue(kk, j, b):
            pltpu.async_copy(x_hbm.at[ib[j].at[0]], r[b], sg[b])

        def gather_wait(b):
            pltpu.make_async_copy(
                x_hbm.at[ib0.at[0]], r[b], sg[b]).wait()

        def scatter(j, b):
            pltpu.sync_copy(r[b], acc.at[ib[j].at[1]], add=True)

        for j in range(min(4, K)):
            idx_issue(j, j)

        idx_wait(0)
        gather_issue(0, 0, 0)

        zvec = jnp.zeros((16,), jnp.float32)

        @pl.loop(0, CHUNK)
        def _(i):
            @pl.loop(0, d, step=16)
            def _(j):
                r1[i, pl.ds(j, 16)] = zvec

        astart = sid * rows_per
        zcopies = []
        for i in range(nz):
            zcopies.append(pltpu.async_copy(
                r1, acc.at[pl.ds(astart + i * CHUNK, CHUNK)], sg1))
        if remz:
            zcopies.append(pltpu.async_copy(
                r1.at[pl.ds(0, remz)],
                acc.at[pl.ds(astart + nz * CHUNK, remz)], sg1))
        for cp in zcopies:
            cp.wait()

        if extra:
            @pl.when(sid == 0)
            def _():
                pltpu.sync_copy(r1.at[pl.ds(0, extra)],
                                acc.at[pl.ds(NS * rows_per, extra)])

        if K > 1:
            idx_wait(1)
            gather_issue(1, 1, 1)

        plsc.subcore_barrier()

        U = K // 4

        @pl.loop(0, U)
        def _(u):
            k0 = 4 * u
            for j in range(4):
                kk = k0 + j
                gather_wait(j % 2)
                scatter(j, j % 2)

                @pl.when(kk + 4 < K)
                def _():
                    idx_issue(kk + 4, j)

                @pl.when(kk + 2 < K)
                def _():
                    idx_wait((j + 2) % 4)
                    gather_issue(kk + 2, (j + 2) % 4, j % 2)

        for kk in range(4 * U, K):
            j = kk % 4
            gather_wait(j % 2)
            scatter(j, j % 2)
            if kk + 2 < K:
                idx_wait((j + 2) % 4)
                gather_issue(kk + 2, (j + 2) % 4, j % 2)

        if L:
            @pl.when(wid < L)
            def _():
                pltpu.sync_copy(
                    ei_hbm.at[pl.ds(0, 2),
                              pl.ds((NW * K + wid) * CHUNK, CHUNK)], ib0)
                pltpu.async_copy(x_hbm.at[ib0.at[0]], r0, sg0).wait()
                pltpu.sync_copy(r0, acc.at[ib0.at[1]], add=True)

        plsc.subcore_barrier()
        pltpu.sync_copy(acc.at[pl.ds(astart, rows_per)],
                        out_hbm.at[cid, pl.ds(astart, rows_per)])
        if extra:
            @pl.when(sid == 0)
            def _():
                pltpu.sync_copy(acc.at[pl.ds(NS * rows_per, extra)],
                                out_hbm.at[cid, pl.ds(NS * rows_per, extra)])

    return k(x2d, edge_index)


def _mlp_body(x_ref, a_ref, w1_ref, b1_ref, w2_ref, b2_ref, o_ref):
    h = x_ref[...] + a_ref[0] + a_ref[1]
    h = jnp.dot(h, w1_ref[...], preferred_element_type=jnp.float32)
    h = jnp.maximum(h + b1_ref[...], 0.0)
    h = jnp.dot(h, w2_ref[...], preferred_element_type=jnp.float32)
    o_ref[...] = jnp.maximum(h + b2_ref[...], 0.0)


def _mlp(x2d, aggs, W1, b1, W2, b2):
    n, d = aggs.shape[1], aggs.shape[2]
    rows = 1000
    grid = (n // rows,)
    return pl.pallas_call(
        _mlp_body,
        grid=grid,
        compiler_params=pltpu.CompilerParams(
            dimension_semantics=("parallel",)),
        in_specs=[
            pl.BlockSpec((rows, d), lambda i: (i, 0)),
            pl.BlockSpec((NC, rows, d), lambda i: (0, i, 0)),
            pl.BlockSpec((d, d), lambda i: (0, 0)),
            pl.BlockSpec((1, d), lambda i: (0, 0)),
            pl.BlockSpec((d, d), lambda i: (0, 0)),
            pl.BlockSpec((1, d), lambda i: (0, 0)),
        ],
        out_specs=pl.BlockSpec((rows, d), lambda i: (i, 0)),
        out_shape=jax.ShapeDtypeStruct((n, d), jnp.float32),
    )(x2d, aggs, W1, b1, W2, b2)


def kernel(x, edge_index, W1, b1, W2, b2):
    n = x.shape[0]
    d = x.shape[-1]
    x2d = x.reshape(n, d)
    ei = edge_index.reshape(2, -1)
    aggs = _sc_agg(x2d, ei)
    return _mlp(x2d, aggs, W1, b1.reshape(1, d), W2, b2.reshape(1, d))


# speedup vs baseline: 1.0391x; 1.0391x over previous
import functools

import jax
import jax.numpy as jnp
from jax import lax
from jax.experimental import pallas as pl
from jax.experimental.pallas import tpu as pltpu
from jax.experimental.pallas import tpu_sc as plsc

NC = 2
NS = 16
NW = NC * NS
CHUNK = 128


def _sc_agg(x2d, edge_index):
    n, d = x2d.shape
    e = edge_index.shape[1]
    assert e % CHUNK == 0
    ncht = e // CHUNK
    K = ncht // NW
    L = ncht - NW * K
    rows_per = (n // NS) // 8 * 8
    extra = n - NS * rows_per
    nz = rows_per // CHUNK
    remz = rows_per - nz * CHUNK

    mesh = plsc.VectorSubcoreMesh(core_axis_name="c", subcore_axis_name="s")

    @functools.partial(
        pl.kernel,
        out_type=jax.ShapeDtypeStruct((NC, n, d), jnp.float32),
        mesh=mesh,
        scratch_types=[
            pltpu.VMEM((2, CHUNK), jnp.int32),
            pltpu.VMEM((2, CHUNK), jnp.int32),
            pltpu.VMEM((2, CHUNK), jnp.int32),
            pltpu.VMEM((2, CHUNK), jnp.int32),
            pltpu.VMEM((CHUNK, d), jnp.float32),
            pltpu.VMEM((CHUNK, d), jnp.float32),
            pltpu.VMEM_SHARED((n, d), jnp.float32),
            pltpu.SemaphoreType.DMA,
            pltpu.SemaphoreType.DMA,
            pltpu.SemaphoreType.DMA,
            pltpu.SemaphoreType.DMA,
            pltpu.SemaphoreType.DMA,
            pltpu.SemaphoreType.DMA,
        ],
    )
    def k(x_hbm, ei_hbm, out_hbm,
          ib0, ib1, ib2, ib3, r0, r1, acc,
          sb0, sb1, sb2, sb3, sg0, sg1):
        ib = [ib0, ib1, ib2, ib3]
        sb = [sb0, sb1, sb2, sb3]
        r = [r0, r1]
        sg = [sg0, sg1]
        cid = lax.axis_index("c")
        sid = lax.axis_index("s")
        wid = cid * NS + sid

        def col(kk):
            return (wid + NW * kk) * CHUNK

        def idx_issue(kk, j):
            pltpu.async_copy(
                ei_hbm.at[pl.ds(0, 2), pl.ds(col(kk), CHUNK)], ib[j], sb[j])

        def idx_wait(j):
            pltpu.make_async_copy(
                ei_hbm.at[pl.ds(0, 2), pl.ds(0, CHUNK)], ib[j], sb[j]).wait()

        def gather_issue(kk, j, b):
            pltpu.async_copy(x_hbm.at[ib[j].at[0]], r[b], sg[b])

        def gather_wait(b):
            pltpu.make_async_copy(
                x_hbm.at[ib0.at[0]], r[b], sg[b]).wait()

        def scatter(j, b):
            pltpu.sync_copy(r[b], acc.at[ib[j].at[1]], add=True)

        for j in range(min(4, K)):
            idx_issue(j, j)

        idx_wait(0)
        gather_issue(0, 0, 0)

        zvec = jnp.zeros((16,), jnp.float32)

        @pl.loop(0, CHUNK)
        def _(i):
            @pl.loop(0, d, step=16)
            def _(j):
                r1[i, pl.ds(j, 16)] = zvec

        astart = sid * rows_per
        zcopies = []
        for i in range(nz):
            zcopies.append(pltpu.async_copy(
                r1, acc.at[pl.ds(astart + i * CHUNK, CHUNK)], sg1))
        if remz:
            zcopies.append(pltpu.async_copy(
                r1.at[pl.ds(0, remz)],
                acc.at[pl.ds(astart + nz * CHUNK, remz)], sg1))
        for cp in zcopies:
            cp.wait()

        if extra:
            @pl.when(sid == 0)
            def _():
                pltpu.sync_copy(r1.at[pl.ds(0, extra)],
                                acc.at[pl.ds(NS * rows_per, extra)])

        if K > 1:
            idx_wait(1)
            gather_issue(1, 1, 1)

        plsc.subcore_barrier()

        U = K // 4

        @pl.loop(0, U)
        def _(u):
            k0 = 4 * u
            for j in range(4):
                kk = k0 + j
                gather_wait(j % 2)
                scatter(j, j % 2)

                @pl.when(kk + 4 < K)
                def _():
                    idx_issue(kk + 4, j)

                @pl.when(kk + 2 < K)
                def _():
                    idx_wait((j + 2) % 4)
                    gather_issue(kk + 2, (j + 2) % 4, j % 2)

        for kk in range(4 * U, K):
            j = kk % 4
            gather_wait(j % 2)
            scatter(j, j % 2)
            if kk + 2 < K:
                idx_wait((j + 2) % 4)
                gather_issue(kk + 2, (j + 2) % 4, j % 2)

        if L:
            @pl.when(wid < L)
            def _():
                pltpu.sync_copy(
                    ei_hbm.at[pl.ds(0, 2),
                              pl.ds((NW * K + wid) * CHUNK, CHUNK)], ib0)
                pltpu.async_copy(x_hbm.at[ib0.at[0]], r0, sg0).wait()
                pltpu.sync_copy(r0, acc.at[ib0.at[1]], add=True)

        plsc.subcore_barrier()
        pltpu.sync_copy(acc.at[pl.ds(astart, rows_per)],
                        out_hbm.at[cid, pl.ds(astart, rows_per)])
        if extra:
            @pl.when(sid == 0)
            def _():
                pltpu.sync_copy(acc.at[pl.ds(NS * rows_per, extra)],
                                out_hbm.at[cid, pl.ds(NS * rows_per, extra)])

    return k(x2d, edge_index)


def _mlp_body(x_ref, a_ref, w1_ref, b1_ref, w2_ref, b2_ref, o_ref):
    h = x_ref[...] + a_ref[0] + a_ref[1]
    h = jnp.dot(h, w1_ref[...], preferred_element_type=jnp.float32)
    h = jnp.maximum(h + b1_ref[...], 0.0)
    h = jnp.dot(h, w2_ref[...], preferred_element_type=jnp.float32)
    o_ref[...] = jnp.maximum(h + b2_ref[...], 0.0)


def _mlp(x2d, aggs, W1, b1, W2, b2):
    n, d = aggs.shape[1], aggs.shape[2]
    rows = 5000
    grid = (n // rows,)
    return pl.pallas_call(
        _mlp_body,
        grid=grid,
        compiler_params=pltpu.CompilerParams(
            dimension_semantics=("parallel",)),
        in_specs=[
            pl.BlockSpec((rows, d), lambda i: (i, 0)),
            pl.BlockSpec((NC, rows, d), lambda i: (0, i, 0)),
            pl.BlockSpec((d, d), lambda i: (0, 0)),
            pl.BlockSpec((1, d), lambda i: (0, 0)),
            pl.BlockSpec((d, d), lambda i: (0, 0)),
            pl.BlockSpec((1, d), lambda i: (0, 0)),
        ],
        out_specs=pl.BlockSpec((rows, d), lambda i: (i, 0)),
        out_shape=jax.ShapeDtypeStruct((n, d), jnp.float32),
    )(x2d, aggs, W1, b1, W2, b2)


def kernel(x, edge_index, W1, b1, W2, b2):
    n = x.shape[0]
    d = x.shape[-1]
    x2d = x.reshape(n, d)
    ei = edge_index.reshape(2, -1)
    aggs = _sc_agg(x2d, ei)
    return _mlp(x2d, aggs, W1, b1.reshape(1, d), W2, b2.reshape(1, d))
